# baseline (speedup 1.0000x reference)
import functools

import jax
import jax.numpy as jnp
from jax import lax
from jax.experimental import pallas as pl
from jax.experimental.pallas import tpu as pltpu

D = 768
H = 2048
E = 8
NUM_NULL = 8
RHO = 0.5
LANES = 128
NEG = -1e30


def _gate_body(x_ref, gw_ref, bias_ref, coeff_ref, aux_ref, acc_ref, sacc_ref):
    i = pl.program_id(0)
    n_tok_total = pl.num_programs(0) * x_ref.shape[0]

    x = x_ref[...]
    l = lax.dot_general(x, gw_ref[...], (((1,), (1,)), ((), ())),
                        preferred_element_type=jnp.float32)
    l = l + bias_ref[...]
    it = lax.broadcasted_iota(jnp.int32, l.shape, 1)
    real_mask = it < E
    null_col = it == E
    slot_mask = it <= E
    l = jnp.where(slot_mask, l, NEG)

    m_all = jnp.max(l, axis=1, keepdims=True)
    ez = jnp.exp(l - m_all)
    z = jnp.sum(ez * jnp.where(null_col, 8.0, 1.0) * slot_mask, axis=1, keepdims=True)
    p = ez / z
    p_null = jnp.sum(jnp.where(null_col, p, 0.0), axis=1, keepdims=True)

    p_real = jnp.where(real_mask, p, -1.0)
    max1 = jnp.max(p_real, axis=1, keepdims=True)
    idx1 = jnp.min(jnp.where((p_real == max1) & real_mask, it, LANES),
                   axis=1, keepdims=True)
    pick1_null = p_null > max1

    p_excl = jnp.where(it == idx1, -1.0, p_real)
    max2r = jnp.max(p_excl, axis=1, keepdims=True)
    idx2r = jnp.min(jnp.where((p_excl == max2r) & real_mask, it, LANES),
                    axis=1, keepdims=True)
    max2 = jnp.where(pick1_null, max1, max2r)
    idx2 = jnp.where(pick1_null, idx1, idx2r)
    pick2_null = p_null > max2

    w1 = jnp.where(pick1_null, 0.0, max1)
    w2 = jnp.where(pick2_null, 0.0, max2)
    s = jnp.clip(w1 + w2, 1e-6, None)
    w1n = w1 / s
    w2n = w2 / s

    one1 = ((it == idx1) & jnp.logical_not(pick1_null)).astype(jnp.float32)
    one2 = ((it == idx2) & jnp.logical_not(pick2_null)).astype(jnp.float32)
    coeff_ref[...] = w1n * one1 + w2n * one2

    mr = jnp.max(jnp.where(real_mask, l, NEG), axis=1, keepdims=True)
    er = jnp.exp(l - mr) * real_mask
    pr = er / jnp.sum(er, axis=1, keepdims=True)

    p_sum = jnp.sum(pr, axis=0)
    cnt = jnp.sum(one1 + one2, axis=0)
    nullcnt = jnp.sum(pick1_null.astype(jnp.float32) + pick2_null.astype(jnp.float32))
    lse = m_all + jnp.log(z)
    lse2 = jnp.sum(lse * lse)

    @pl.when(i == 0)
    def _init():
        acc_ref[...] = jnp.zeros_like(acc_ref)
        sacc_ref[0] = 0.0
        sacc_ref[1] = 0.0

    acc_ref[0, :] += p_sum
    acc_ref[1, :] += cnt
    sacc_ref[0] += nullcnt
    sacc_ref[1] += lse2

    @pl.when(i == pl.num_programs(0) - 1)
    def _fin():
        lane_real = lax.broadcasted_iota(jnp.int32, (1, LANES), 1) < E
        p_mean = jnp.where(lane_real, acc_ref[0:1, :], 0.0) / n_tok_total
        counts = jnp.where(lane_real, acc_ref[1:2, :], 0.0)
        total = jnp.clip(jnp.sum(counts), 1e-6, None)
        l_bal = E * jnp.sum(counts / total * p_mean)
        null_rate = sacc_ref[0] / (n_tok_total * 2.0)
        l_null = (null_rate - RHO) ** 2
        l_z = sacc_ref[1] / n_tok_total
        aux_ref[0, 0] = 0.02 * l_bal + 0.001 * l_z + 0.01 * l_null


def _gate(x, gw_pad, bias_pad, block_t=256):
    n = x.shape[0]
    grid = n // block_t
    return pl.pallas_call(
        _gate_body,
        grid=(grid,),
        in_specs=[
            pl.BlockSpec((block_t, D), lambda i: (i, 0)),
            pl.BlockSpec((LANES, D), lambda i: (0, 0)),
            pl.BlockSpec((1, LANES), lambda i: (0, 0)),
        ],
        out_specs=[
            pl.BlockSpec((block_t, LANES), lambda i: (i, 0)),
            pl.BlockSpec(memory_space=pltpu.SMEM),
        ],
        out_shape=[
            jax.ShapeDtypeStruct((n, LANES), jnp.float32),
            jax.ShapeDtypeStruct((1, 1), jnp.float32),
        ],
        scratch_shapes=[
            pltpu.VMEM((8, LANES), jnp.float32),
            pltpu.SMEM((2,), jnp.float32),
        ],
    )(x, gw_pad, bias_pad)


def _ffn_body(x_ref, w1_ref, w2_ref, w3_ref, coeff_ref, out_ref, acc_ref):
    e = pl.program_id(0)
    tb = pl.program_id(1)
    bt = x_ref.shape[0]

    x = x_ref[...]
    hg = jnp.dot(x, w1_ref[0], preferred_element_type=jnp.float32)
    hu = jnp.dot(x, w2_ref[0], preferred_element_type=jnp.float32)
    h = (hg * jax.nn.sigmoid(hg)) * hu
    part = jnp.dot(h, w3_ref[0], preferred_element_type=jnp.float32)

    it = lax.broadcasted_iota(jnp.int32, coeff_ref.shape, 1)
    sel = jnp.sum(jnp.where(it == e - 1, coeff_ref[...], 0.0), axis=1, keepdims=True)
    scale = jnp.where(e == 0, 1.0, sel)
    part = part * scale

    sl = pl.ds(tb * bt, bt)

    @pl.when(e == 0)
    def _init():
        acc_ref[sl, :] = part

    @pl.when(e != 0)
    def _acc():
        acc_ref[sl, :] += part

    out_ref[...] = acc_ref[sl, :]


def _ffn(x, w1s, w2s, w3s, coeff, block_t=256):
    n = x.shape[0]
    grid = (E + 1, n // block_t)
    return pl.pallas_call(
        _ffn_body,
        grid=grid,
        in_specs=[
            pl.BlockSpec((block_t, D), lambda e, tb: (tb, 0)),
            pl.BlockSpec((1, D, H), lambda e, tb: (e, 0, 0)),
            pl.BlockSpec((1, D, H), lambda e, tb: (e, 0, 0)),
            pl.BlockSpec((1, H, D), lambda e, tb: (e, 0, 0)),
            pl.BlockSpec((block_t, LANES), lambda e, tb: (tb, 0)),
        ],
        out_specs=pl.BlockSpec((block_t, D), lambda e, tb: (tb, 0)),
        out_shape=jax.ShapeDtypeStruct((n, D), jnp.float32),
        scratch_shapes=[pltpu.VMEM((n, D), jnp.float32)],
    )(x, w1s, w2s, w3s, coeff)


def kernel(x, gate_w, logit_bias, null_logit, W_gate, W_up, W_down,
           Ws_gate, Ws_up, Ws_down):
    b, t, d = x.shape
    n = b * t
    xf = x.reshape(n, d)

    gw_pad = jnp.zeros((LANES, D), jnp.float32).at[:E].set(gate_w)
    bias_pad = (jnp.zeros((1, LANES), jnp.float32)
                .at[0, :E].set(logit_bias)
                .at[0, E].set(null_logit))

    coeff, aux = _gate(xf, gw_pad, bias_pad)

    w1s = jnp.concatenate([Ws_gate.T[None], W_gate], axis=0)
    w2s = jnp.concatenate([Ws_up.T[None], W_up], axis=0)
    w3s = jnp.concatenate([Ws_down.T[None], W_down], axis=0)

    y = _ffn(xf, w1s, w2s, w3s, coeff)
    return (y.reshape(b, t, d), aux[0, 0])


# speedup vs baseline: 1.1997x; 1.1997x over previous
import functools

import jax
import jax.numpy as jnp
from jax import lax
from jax.experimental import pallas as pl
from jax.experimental.pallas import tpu as pltpu

D = 768
H = 2048
E = 8
RHO = 0.5
LANES = 128
NEG = -1e30
N_TOK = 2048
BLK = 256
NBPE = N_TOK // BLK
MAXB = (2 * N_TOK) // BLK + E - 1
NSLOT = E * N_TOK


def _gate_body(x_ref, gw_ref, bias_ref, coeff_ref, coefft_ref, meta_ref,
               aux_ref, acc_ref, sacc_ref):
    i = pl.program_id(0)
    n_tok_total = pl.num_programs(0) * x_ref.shape[0]

    x = x_ref[...]
    l = lax.dot_general(x, gw_ref[...], (((1,), (1,)), ((), ())),
                        preferred_element_type=jnp.float32)
    l = l + bias_ref[...]
    it = lax.broadcasted_iota(jnp.int32, l.shape, 1)
    real_mask = it < E
    null_col = it == E
    slot_mask = it <= E
    l = jnp.where(slot_mask, l, NEG)

    m_all = jnp.max(l, axis=1, keepdims=True)
    ez = jnp.exp(l - m_all)
    z = jnp.sum(ez * jnp.where(null_col, 8.0, 1.0) * slot_mask, axis=1, keepdims=True)
    p = ez / z
    p_null = jnp.sum(jnp.where(null_col, p, 0.0), axis=1, keepdims=True)

    p_real = jnp.where(real_mask, p, -1.0)
    max1 = jnp.max(p_real, axis=1, keepdims=True)
    idx1 = jnp.min(jnp.where((p_real == max1) & real_mask, it, LANES),
                   axis=1, keepdims=True)
    pick1_null = p_null > max1

    p_excl = jnp.where(it == idx1, -1.0, p_real)
    max2r = jnp.max(p_excl, axis=1, keepdims=True)
    idx2r = jnp.min(jnp.where((p_excl == max2r) & real_mask, it, LANES),
                    axis=1, keepdims=True)
    max2 = jnp.where(pick1_null, max1, max2r)
    idx2 = jnp.where(pick1_null, idx1, idx2r)
    pick2_null = p_null > max2

    w1 = jnp.where(pick1_null, 0.0, max1)
    w2 = jnp.where(pick2_null, 0.0, max2)
    s = jnp.clip(w1 + w2, 1e-6, None)
    w1n = w1 / s
    w2n = w2 / s

    one1 = ((it == idx1) & jnp.logical_not(pick1_null)).astype(jnp.float32)
    one2 = ((it == idx2) & jnp.logical_not(pick2_null)).astype(jnp.float32)
    coeff = w1n * one1 + w2n * one2
    coeff_ref[...] = coeff
    coefft_ref[...] = coeff[:, :E].T

    e1f = jnp.where(pick1_null, 0.0, idx1.astype(jnp.float32))
    e2f = jnp.where(pick2_null, 0.0, idx2.astype(jnp.float32))
    w1o = jnp.where(pick1_null, 0.0, w1n)
    w2o = jnp.where(pick2_null, 0.0, w2n)
    bt = x.shape[0]
    meta_ref[...] = jnp.concatenate(
        [e1f.T, e2f.T, w1o.T, w2o.T,
         jnp.zeros((4, bt), jnp.float32)], axis=0)

    mr = jnp.max(jnp.where(real_mask, l, NEG), axis=1, keepdims=True)
    er = jnp.exp(l - mr) * real_mask
    pr = er / jnp.sum(er, axis=1, keepdims=True)

    p_sum = jnp.sum(pr, axis=0)
    cnt = jnp.sum(one1 + one2, axis=0)
    nullcnt = jnp.sum(pick1_null.astype(jnp.float32) + pick2_null.astype(jnp.float32))
    lse = m_all + jnp.log(z)
    lse2 = jnp.sum(lse * lse)

    @pl.when(i == 0)
    def _init():
        acc_ref[...] = jnp.zeros_like(acc_ref)
        sacc_ref[0] = 0.0
        sacc_ref[1] = 0.0

    acc_ref[0, :] += p_sum
    acc_ref[1, :] += cnt
    sacc_ref[0] += nullcnt
    sacc_ref[1] += lse2

    @pl.when(i == pl.num_programs(0) - 1)
    def _fin():
        lane_real = lax.broadcasted_iota(jnp.int32, (1, LANES), 1) < E
        p_mean = jnp.where(lane_real, acc_ref[0:1, :], 0.0) / n_tok_total
        counts = jnp.where(lane_real, acc_ref[1:2, :], 0.0)
        total = jnp.clip(jnp.sum(counts), 1e-6, None)
        l_bal = E * jnp.sum(counts / total * p_mean)
        null_rate = sacc_ref[0] / (n_tok_total * 2.0)
        l_null = (null_rate - RHO) ** 2
        l_z = sacc_ref[1] / n_tok_total
        aux_ref[0, 0] = 0.02 * l_bal + 0.001 * l_z + 0.01 * l_null


def _gate(x, gw_pad, bias_pad, block_t=256):
    n = x.shape[0]
    grid = n // block_t
    return pl.pallas_call(
        _gate_body,
        grid=(grid,),
        in_specs=[
            pl.BlockSpec((block_t, D), lambda i: (i, 0)),
            pl.BlockSpec((LANES, D), lambda i: (0, 0)),
            pl.BlockSpec((1, LANES), lambda i: (0, 0)),
        ],
        out_specs=[
            pl.BlockSpec((block_t, LANES), lambda i: (i, 0)),
            pl.BlockSpec((E, block_t), lambda i: (0, i)),
            pl.BlockSpec((8, block_t), lambda i: (0, i)),
            pl.BlockSpec(memory_space=pltpu.SMEM),
        ],
        out_shape=[
            jax.ShapeDtypeStruct((n, LANES), jnp.float32),
            jax.ShapeDtypeStruct((E, n), jnp.float32),
            jax.ShapeDtypeStruct((8, n), jnp.float32),
            jax.ShapeDtypeStruct((1, 1), jnp.float32),
        ],
        scratch_shapes=[
            pltpu.VMEM((8, LANES), jnp.float32),
            pltpu.SMEM((2,), jnp.float32),
        ],
    )(x, gw_pad, bias_pad)


def _grouped_body(bm_ref, xs_ref, w1_ref, w2_ref, w3_ref, out_ref):
    b = pl.program_id(0)
    active = bm_ref[2 * 32 + b]

    @pl.when(active == 1)
    def _go():
        x = xs_ref[...]
        hg = jnp.dot(x, w1_ref[0], preferred_element_type=jnp.float32)
        hu = jnp.dot(x, w2_ref[0], preferred_element_type=jnp.float32)
        h = (hg * jax.nn.sigmoid(hg)) * hu
        out_ref[...] = jnp.dot(h, w3_ref[0], preferred_element_type=jnp.float32)


def _grouped(blockmap, xs, W_gate, W_up, W_down):
    grid_spec = pltpu.PrefetchScalarGridSpec(
        num_scalar_prefetch=1,
        grid=(MAXB,),
        in_specs=[
            pl.BlockSpec((BLK, D),
                         lambda b, bm: (bm[b] * NBPE + bm[32 + b], 0)),
            pl.BlockSpec((1, D, H), lambda b, bm: (bm[b], 0, 0)),
            pl.BlockSpec((1, D, H), lambda b, bm: (bm[b], 0, 0)),
            pl.BlockSpec((1, H, D), lambda b, bm: (bm[b], 0, 0)),
        ],
        out_specs=pl.BlockSpec((BLK, D),
                               lambda b, bm: (bm[b] * NBPE + bm[32 + b], 0)),
    )
    return pl.pallas_call(
        _grouped_body,
        grid_spec=grid_spec,
        out_shape=jax.ShapeDtypeStruct((NSLOT, D), jnp.float32),
    )(blockmap, xs, W_gate, W_up, W_down)


def _shared_body(x_ref, w1_ref, w2_ref, w3_ref, out_ref):
    x = x_ref[...]
    hg = jnp.dot(x, w1_ref[...], preferred_element_type=jnp.float32)
    hu = jnp.dot(x, w2_ref[...], preferred_element_type=jnp.float32)
    h = (hg * jax.nn.sigmoid(hg)) * hu
    out_ref[...] = jnp.dot(h, w3_ref[...], preferred_element_type=jnp.float32)


def _shared(x, w1, w2, w3, block_t=256):
    n = x.shape[0]
    return pl.pallas_call(
        _shared_body,
        grid=(n // block_t,),
        in_specs=[
            pl.BlockSpec((block_t, D), lambda i: (i, 0)),
            pl.BlockSpec((D, H), lambda i: (0, 0)),
            pl.BlockSpec((D, H), lambda i: (0, 0)),
            pl.BlockSpec((H, D), lambda i: (0, 0)),
        ],
        out_specs=pl.BlockSpec((block_t, D), lambda i: (i, 0)),
        out_shape=jax.ShapeDtypeStruct((n, D), jnp.float32),
    )(x, w1, w2, w3)


def _routing_host(coefft, meta, x):
    assigned = (coefft > 0.0)
    ai = assigned.astype(jnp.int32)
    ranks = jnp.cumsum(ai, axis=1) - ai
    counts = jnp.sum(ai, axis=1)
    tokn = jnp.broadcast_to(jnp.arange(N_TOK, dtype=jnp.int32)[None], (E, N_TOK))
    tok = jnp.zeros((E, N_TOK), jnp.int32)
    tok = tok.at[jnp.arange(E)[:, None], jnp.where(assigned, ranks, N_TOK - 1)].max(
        jnp.where(assigned, tokn, 0), mode="drop")
    nb = (counts + BLK - 1) // BLK
    pnb = jnp.concatenate([jnp.zeros((1,), jnp.int32), jnp.cumsum(nb)])
    nact = pnb[E]
    bvec = jnp.arange(32, dtype=jnp.int32)
    beff = jnp.clip(jnp.minimum(bvec, nact - 1), 0, None)
    bm_e = jnp.sum((beff[:, None] >= pnb[None, 1:]).astype(jnp.int32), axis=1)
    bm_e = jnp.minimum(bm_e, E - 1)
    bm_cb = jnp.clip(beff - pnb[bm_e], 0, NBPE - 1)
    bm_act = (bvec < nact).astype(jnp.int32)
    blockmap = jnp.concatenate([bm_e, bm_cb, bm_act]).astype(jnp.int32)
    xs = x[tok.reshape(-1)]
    return counts, blockmap, tok, ranks, xs


def _combine_host(shared_out, ys, meta, ranks):
    n = shared_out.shape[0]
    nvec = jnp.arange(n, dtype=jnp.int32)
    e1 = meta[0].astype(jnp.int32)
    e2 = meta[1].astype(jnp.int32)
    w1 = meta[2]
    w2 = meta[3]
    rk = ranks.reshape(-1)
    s1 = e1 * N_TOK + rk[e1 * N_TOK + nvec]
    s2 = e2 * N_TOK + rk[e2 * N_TOK + nvec]
    r1 = ys[s1]
    r2 = ys[s2]
    y = shared_out
    y = y + jnp.where((w1 > 0)[:, None], w1[:, None] * r1, 0.0)
    y = y + jnp.where((w2 > 0)[:, None], w2[:, None] * r2, 0.0)
    return y


def kernel(x, gate_w, logit_bias, null_logit, W_gate, W_up, W_down,
           Ws_gate, Ws_up, Ws_down):
    b, t, d = x.shape
    n = b * t
    xf = x.reshape(n, d)

    gw_pad = jnp.zeros((LANES, D), jnp.float32).at[:E].set(gate_w)
    bias_pad = (jnp.zeros((1, LANES), jnp.float32)
                .at[0, :E].set(logit_bias)
                .at[0, E].set(null_logit))

    coeff, coefft, meta, aux = _gate(xf, gw_pad, bias_pad)

    counts, blockmap, tok, ranks, xs = _routing_host(coefft, meta, xf)
    ys = _grouped(blockmap, xs, W_gate, W_up, W_down)
    shared_out = _shared(xf, Ws_gate.T, Ws_up.T, Ws_down.T)
    y = _combine_host(shared_out, ys, meta, ranks)
    return (y.reshape(b, t, d), aux[0, 0])


# speedup vs baseline: 2.0579x; 1.7152x over previous
import functools

import jax
import jax.numpy as jnp
from jax import lax
from jax.experimental import pallas as pl
from jax.experimental.pallas import tpu as pltpu
from jax.experimental.pallas import tpu_sc as plsc

D = 768
H = 2048
E = 8
RHO = 0.5
LANES = 128
NEG = -1e30
N = 2048
BLK = 256
NBPE = N // BLK
MAXB = (2 * N) // BLK + E - 1
NSLOT = E * N
NXS = NSLOT + 256


def _gate_body(x_ref, gw_ref, bias_ref, sidx_ref, meta_ref, bm_ref, aux_ref,
               acc_ref, sacc_ref):
    i = pl.program_id(0)
    bt = x_ref.shape[0]
    n_tok_total = pl.num_programs(0) * bt

    x = x_ref[...]
    l = lax.dot_general(x, gw_ref[...], (((1,), (1,)), ((), ())),
                        preferred_element_type=jnp.float32)
    l = l + bias_ref[...]
    it = lax.broadcasted_iota(jnp.int32, l.shape, 1)
    real_mask = it < E
    null_col = it == E
    slot_mask = it <= E
    l = jnp.where(slot_mask, l, NEG)

    m_all = jnp.max(l, axis=1, keepdims=True)
    ez = jnp.exp(l - m_all)
    z = jnp.sum(ez * jnp.where(null_col, 8.0, 1.0) * slot_mask, axis=1, keepdims=True)
    p = ez / z
    p_null = jnp.sum(jnp.where(null_col, p, 0.0), axis=1, keepdims=True)

    p_real = jnp.where(real_mask, p, -1.0)
    max1 = jnp.max(p_real, axis=1, keepdims=True)
    idx1 = jnp.min(jnp.where((p_real == max1) & real_mask, it, LANES),
                   axis=1, keepdims=True)
    pick1_null = p_null > max1

    p_excl = jnp.where(it == idx1, -1.0, p_real)
    max2r = jnp.max(p_excl, axis=1, keepdims=True)
    idx2r = jnp.min(jnp.where((p_excl == max2r) & real_mask, it, LANES),
                    axis=1, keepdims=True)
    max2 = jnp.where(pick1_null, max1, max2r)
    idx2 = jnp.where(pick1_null, idx1, idx2r)
    pick2_null = p_null > max2

    w1 = jnp.where(pick1_null, 0.0, max1)
    w2 = jnp.where(pick2_null, 0.0, max2)
    s = jnp.clip(w1 + w2, 1e-6, None)
    w1n = w1 / s
    w2n = w2 / s

    one1 = ((it == idx1) & jnp.logical_not(pick1_null)).astype(jnp.float32)
    one2 = ((it == idx2) & jnp.logical_not(pick2_null)).astype(jnp.float32)

    @pl.when(i == 0)
    def _init():
        acc_ref[...] = jnp.zeros_like(acc_ref)
        sacc_ref[0] = 0.0
        sacc_ref[1] = 0.0

    assigned = one1 + one2
    carry = acc_ref[1:2, :]
    rit = lax.broadcasted_iota(jnp.int32, (bt, bt), 0)
    cit = lax.broadcasted_iota(jnp.int32, (bt, bt), 1)
    tri = (cit < rit).astype(jnp.float32)
    cum_excl = jnp.dot(tri, assigned, preferred_element_type=jnp.float32)
    rank = cum_excl + carry

    rowiota = lax.broadcasted_iota(jnp.int32, (bt, 1), 0)
    rank1 = jnp.sum(rank * one1, axis=1, keepdims=True).astype(jnp.int32)
    rank2 = jnp.sum(rank * one2, axis=1, keepdims=True).astype(jnp.int32)
    slot1 = jnp.where(pick1_null, NSLOT + rowiota, idx1 * N + rank1)
    slot2 = jnp.where(pick2_null, NSLOT + rowiota, idx2 * N + rank2)
    sidx_ref[...] = jnp.concatenate([slot1.T, slot2.T], axis=0)

    v0 = jnp.logical_not(pick1_null).astype(jnp.float32)
    v1 = jnp.logical_not(pick2_null).astype(jnp.float32)
    w1o = jnp.where(pick1_null, 0.0, w1n)
    w2o = jnp.where(pick2_null, 0.0, w2n)
    meta_ref[...] = jnp.concatenate(
        [v0.T, v1.T, w1o.T, w2o.T, jnp.zeros((4, bt), jnp.float32)], axis=0)

    mr = jnp.max(jnp.where(real_mask, l, NEG), axis=1, keepdims=True)
    er = jnp.exp(l - mr) * real_mask
    pr = er / jnp.sum(er, axis=1, keepdims=True)

    p_sum = jnp.sum(pr, axis=0)
    cnt = jnp.sum(assigned, axis=0)
    nullcnt = jnp.sum(pick1_null.astype(jnp.float32) + pick2_null.astype(jnp.float32))
    lse = m_all + jnp.log(z)
    lse2 = jnp.sum(lse * lse)

    acc_ref[0, :] += p_sum
    acc_ref[1, :] += cnt
    sacc_ref[0] += nullcnt
    sacc_ref[1] += lse2

    @pl.when(i == pl.num_programs(0) - 1)
    def _fin():
        it1 = lax.broadcasted_iota(jnp.int32, (1, LANES), 1)
        lane_real = (it1 < E).astype(jnp.float32)
        counts = acc_ref[1:2, :] * lane_real

        p_mean = acc_ref[0:1, :] * lane_real / n_tok_total
        total = jnp.clip(jnp.sum(counts), 1e-6, None)
        l_bal = E * jnp.sum(counts / total * p_mean)
        null_rate = sacc_ref[0] / (n_tok_total * 2.0)
        l_null = (null_rate - RHO) ** 2
        l_z = sacc_ref[1] / n_tok_total
        aux_ref[0, 0] = 0.02 * l_bal + 0.001 * l_z + 0.01 * l_null

        nb = jnp.floor((counts + (BLK - 1)) * (1.0 / BLK)) * lane_real
        i2a = lax.broadcasted_iota(jnp.int32, (LANES, LANES), 0)
        i2b = lax.broadcasted_iota(jnp.int32, (LANES, LANES), 1)
        mtri = (i2a < i2b).astype(jnp.float32)
        pexcl = lax.dot_general(nb, mtri, (((1,), (0,)), ((), ())),
                                preferred_element_type=jnp.float32)
        pincl = pexcl + nb
        nact = jnp.sum(nb)
        bb = (it1 & 31).astype(jnp.float32)
        beff = jnp.clip(jnp.minimum(bb, nact - 1.0), 0.0, None)
        evec = jnp.zeros_like(bb)
        for ee in range(E):
            s_incl = jnp.sum(pincl * (it1 == ee))
            evec = evec + (beff >= s_incl).astype(jnp.float32)
        evec = jnp.minimum(evec, float(E - 1))
        pj = jnp.zeros_like(bb)
        for ee in range(E):
            s_excl = jnp.sum(pexcl * (it1 == ee))
            pj = pj + jnp.where(evec == ee, s_excl, 0.0)
        cb = jnp.clip(beff - pj, 0.0, float(NBPE - 1))
        act = (bb < nact).astype(jnp.float32)
        bm_f = jnp.where(it1 < 32, evec,
                         jnp.where(it1 < 64, cb,
                                   jnp.where(it1 < 96, act, 0.0)))
        bm_ref[...] = bm_f.astype(jnp.int32)


def _gate(x, gw_pad, bias_pad, block_t=256):
    n = x.shape[0]
    return pl.pallas_call(
        _gate_body,
        grid=(n // block_t,),
        in_specs=[
            pl.BlockSpec((block_t, D), lambda i: (i, 0)),
            pl.BlockSpec((LANES, D), lambda i: (0, 0)),
            pl.BlockSpec((1, LANES), lambda i: (0, 0)),
        ],
        out_specs=[
            pl.BlockSpec((2, block_t), lambda i: (0, i)),
            pl.BlockSpec((8, block_t), lambda i: (0, i)),
            pl.BlockSpec((1, LANES), lambda i: (0, 0)),
            pl.BlockSpec(memory_space=pltpu.SMEM),
        ],
        out_shape=[
            jax.ShapeDtypeStruct((2, n), jnp.int32),
            jax.ShapeDtypeStruct((8, n), jnp.float32),
            jax.ShapeDtypeStruct((1, LANES), jnp.int32),
            jax.ShapeDtypeStruct((1, 1), jnp.float32),
        ],
        scratch_shapes=[
            pltpu.VMEM((8, LANES), jnp.float32),
            pltpu.SMEM((2,), jnp.float32),
        ],
    )(x, gw_pad, bias_pad)


_SC_MESH = plsc.VectorSubcoreMesh(core_axis_name="c", subcore_axis_name="s")
_SC_PARAMS = pltpu.CompilerParams(needs_layout_passes=False)


def _dispatch_body(x_hbm, sidx_hbm, xs_hbm, idx_v, rows_a, rows_b,
                   sem_a, sem_b, sem_c, sem_d):
    c = lax.axis_index("c")
    s = lax.axis_index("s")
    w = s * 2 + c
    base = w * 128
    tokbase = (w & 15) * 128
    pltpu.sync_copy(sidx_hbm.at[pl.ds(base, 128)], idx_v)

    def ch(i, _):
        off = i * 32
        ia = idx_v[pl.ds(off, 16)]
        ib = idx_v[pl.ds(off + 16, 16)]
        da = pltpu.async_copy(x_hbm.at[pl.ds(tokbase + off, 16)], rows_a, sem_a)
        db = pltpu.async_copy(x_hbm.at[pl.ds(tokbase + off + 16, 16)], rows_b,
                              sem_b)
        da.wait()
        wa = pltpu.async_copy(rows_a, xs_hbm.at[ia], sem_c)
        db.wait()
        wb = pltpu.async_copy(rows_b, xs_hbm.at[ib], sem_d)
        wa.wait()
        wb.wait()
        return 0
    lax.fori_loop(0, 4, ch, 0)


def _dispatch(x, sidx):
    f = pl.kernel(
        _dispatch_body,
        mesh=_SC_MESH,
        compiler_params=_SC_PARAMS,
        out_type=[jax.ShapeDtypeStruct((NXS, D), jnp.float32)],
        scratch_types=[
            pltpu.VMEM((128,), jnp.int32),
            pltpu.VMEM((16, D), jnp.float32),
            pltpu.VMEM((16, D), jnp.float32),
            pltpu.SemaphoreType.DMA,
            pltpu.SemaphoreType.DMA,
            pltpu.SemaphoreType.DMA,
            pltpu.SemaphoreType.DMA,
        ],
    )
    (xs,) = f(x, sidx)
    return xs


def _grouped_body(bm_ref, xs_ref, w1_ref, w2_ref, w3_ref, out_ref):
    b = pl.program_id(0)
    active = bm_ref[2 * 32 + b]

    @pl.when(active == 1)
    def _go():
        x = xs_ref[...]
        hg = jnp.dot(x, w1_ref[0], preferred_element_type=jnp.float32)
        hu = jnp.dot(x, w2_ref[0], preferred_element_type=jnp.float32)
        h = (hg * jax.nn.sigmoid(hg)) * hu
        out_ref[...] = jnp.dot(h, w3_ref[0], preferred_element_type=jnp.float32)


def _grouped(blockmap, xs, W_gate, W_up, W_down):
    grid_spec = pltpu.PrefetchScalarGridSpec(
        num_scalar_prefetch=1,
        grid=(MAXB,),
        in_specs=[
            pl.BlockSpec((BLK, D),
                         lambda b, bm: (bm[b] * NBPE + bm[32 + b], 0)),
            pl.BlockSpec((1, D, H), lambda b, bm: (bm[b], 0, 0)),
            pl.BlockSpec((1, D, H), lambda b, bm: (bm[b], 0, 0)),
            pl.BlockSpec((1, H, D), lambda b, bm: (bm[b], 0, 0)),
        ],
        out_specs=pl.BlockSpec((BLK, D),
                               lambda b, bm: (bm[b] * NBPE + bm[32 + b], 0)),
    )
    return pl.pallas_call(
        _grouped_body,
        grid_spec=grid_spec,
        out_shape=jax.ShapeDtypeStruct((NXS, D), jnp.float32),
    )(blockmap, xs, W_gate, W_up, W_down)


_DNT = (((1,), (1,)), ((), ()))


def _shared_body(x_ref, w1_ref, w2_ref, w3_ref, out_ref):
    x = x_ref[...]
    hg = lax.dot_general(x, w1_ref[...], _DNT,
                         preferred_element_type=jnp.float32)
    hu = lax.dot_general(x, w2_ref[...], _DNT,
                         preferred_element_type=jnp.float32)
    h = (hg * jax.nn.sigmoid(hg)) * hu
    out_ref[...] = lax.dot_general(h, w3_ref[...], _DNT,
                                   preferred_element_type=jnp.float32)


def _shared(x, w1, w2, w3, block_t=256):
    n = x.shape[0]
    return pl.pallas_call(
        _shared_body,
        grid=(n // block_t,),
        in_specs=[
            pl.BlockSpec((block_t, D), lambda i: (i, 0)),
            pl.BlockSpec((H, D), lambda i: (0, 0)),
            pl.BlockSpec((H, D), lambda i: (0, 0)),
            pl.BlockSpec((D, H), lambda i: (0, 0)),
        ],
        out_specs=pl.BlockSpec((block_t, D), lambda i: (i, 0)),
        out_shape=jax.ShapeDtypeStruct((n, D), jnp.float32),
    )(x, w1, w2, w3)


def _combine_body(ys_hbm, sidx_hbm, ya_hbm, idx_v, rows_a, rows_b,
                  sem_a, sem_b, sem_c, sem_d):
    c = lax.axis_index("c")
    s = lax.axis_index("s")
    w = s * 2 + c
    base = w * 128
    pltpu.sync_copy(sidx_hbm.at[pl.ds(base, 128)], idx_v)

    def ch(i, _):
        off = i * 64
        ia = idx_v.at[pl.ds(off, 32)]
        ib = idx_v.at[pl.ds(off + 32, 32)]
        da = pltpu.async_copy(ys_hbm.at[ia], rows_a, sem_a)
        db = pltpu.async_copy(ys_hbm.at[ib], rows_b, sem_b)
        da.wait()
        wa = pltpu.async_copy(rows_a, ya_hbm.at[pl.ds(base + off, 32)], sem_c)
        db.wait()
        wb = pltpu.async_copy(rows_b, ya_hbm.at[pl.ds(base + off + 32, 32)],
                              sem_d)
        wa.wait()
        wb.wait()
        return 0
    lax.fori_loop(0, 2, ch, 0)


def _combine(ys, sidx):
    f = pl.kernel(
        _combine_body,
        mesh=_SC_MESH,
        compiler_params=_SC_PARAMS,
        out_type=[jax.ShapeDtypeStruct((2 * N, D), jnp.float32)],
        scratch_types=[
            pltpu.VMEM((128,), jnp.int32),
            pltpu.VMEM((32, D), jnp.float32),
            pltpu.VMEM((32, D), jnp.float32),
            pltpu.SemaphoreType.DMA,
            pltpu.SemaphoreType.DMA,
            pltpu.SemaphoreType.DMA,
            pltpu.SemaphoreType.DMA,
        ],
    )
    (ya,) = f(ys, sidx)
    return ya


def _add_body(sh_ref, v_ref, a_ref, b_ref, out_ref):
    y = sh_ref[...]
    m0 = v_ref[0:1, :].T > 0.5
    m1 = v_ref[1:2, :].T > 0.5
    w0 = v_ref[2:3, :].T
    w1 = v_ref[3:4, :].T
    y = y + jnp.where(m0, w0 * a_ref[...], 0.0)
    y = y + jnp.where(m1, w1 * b_ref[...], 0.0)
    out_ref[...] = y


def _addk(sh, valid, ya, block_t=256):
    n = sh.shape[0]
    spec = pl.BlockSpec((block_t, D), lambda i: (i, 0))
    spec_hi = pl.BlockSpec((block_t, D), lambda i: (i + N // 256, 0))
    vspec = pl.BlockSpec((8, block_t), lambda i: (0, i))
    return pl.pallas_call(
        _add_body,
        grid=(n // block_t,),
        in_specs=[spec, vspec, spec, spec_hi],
        out_specs=spec,
        out_shape=jax.ShapeDtypeStruct((n, D), jnp.float32),
    )(sh, valid, ya, ya)


def kernel(x, gate_w, logit_bias, null_logit, W_gate, W_up, W_down,
           Ws_gate, Ws_up, Ws_down):
    b, t, d = x.shape
    n = b * t
    xf = x.reshape(n, d)

    gw_pad = jnp.zeros((LANES, D), jnp.float32).at[:E].set(gate_w)
    bias_pad = (jnp.zeros((1, LANES), jnp.float32)
                .at[0, :E].set(logit_bias)
                .at[0, E].set(null_logit))

    sidx, meta, bm, aux = _gate(xf, gw_pad, bias_pad)
    sidx_flat = sidx.reshape(-1)
    xs = _dispatch(xf, sidx_flat)
    ys = _grouped(bm.reshape(LANES), xs, W_gate, W_up, W_down)
    shared_out = _shared(xf, Ws_gate, Ws_up, Ws_down)
    ya = _combine(ys, sidx_flat)
    y = _addk(shared_out, meta, ya)
    return (y.reshape(b, t, d), aux[0, 0])
